# baseline (speedup 1.0000x reference)
import functools

import jax
import jax.numpy as jnp
from jax import lax
from jax.experimental import pallas as pl
from jax.experimental.pallas import tpu as pltpu
from jax.experimental.pallas import tpu_sc as plsc

B = 2048
A = 64
D = 6
R = 16
S = 8
L = 16
NC, NS = 2, 16
NW = NC * NS
NB = B // NW
EW = A * D
EV = EW // L
RW = R * S


def _find_ring_bonds_body(edges_hbm, rings_hbm, out_hbm,
                          e_v, r_v, o_v, ai_v, rb_v, ta_v, tb_v, sem):
    wid = lax.axis_index("c") * NS + lax.axis_index("s")
    base = wid * NB
    edma = pltpu.async_copy(edges_hbm.at[pl.ds(base * EW, NB * EW)], e_v, sem)
    with jax.named_scope("dma_in"):
        pltpu.sync_copy(rings_hbm.at[pl.ds(base * RW, NB * RW)], r_v)

    lane = lax.iota(jnp.int32, L)
    lo_mask = lane < S
    zeros = jnp.zeros((L,), jnp.int32)
    nib = jnp.full((L,), 0x11111111, jnp.int32)

    for v in range(EV):
        ai_v[pl.ds(v * L, L)] = (lane + v * L) // D

    def item(b, carry):
        eoff = pl.multiple_of(b * EW, L)
        roff = pl.multiple_of(b * RW, L)
        for i in range(A // L):
            ta_v[pl.ds(i * L, L)] = zeros
            tb_v[pl.ds(i * L, L)] = zeros
        for r2 in range(0, R, 2):
            idx = r_v[pl.ds(roff + r2 * S, L)]
            rr = r2 % 8
            val = jnp.where(lo_mask, 1 << (4 * rr), 1 << (4 * (rr + 1)))
            tab = ta_v if r2 < 8 else tb_v
            plsc.addupdate_scatter(tab, [idx], val)
        for i in range(A // L):
            va = ta_v[pl.ds(i * L, L)]
            vb = tb_v[pl.ds(i * L, L)]
            va = va | lax.shift_right_logical(va, 1)
            va = (va | lax.shift_right_logical(va, 2)) & nib
            vb = vb | lax.shift_right_logical(vb, 1)
            vb = (vb | lax.shift_right_logical(vb, 2)) & nib
            rb_v[pl.ds(i * L, L)] = va | (vb << 1)
        vals = []
        for v in range(EV):
            nbr_idx = e_v[pl.ds(eoff + v * L, L)]
            self_idx = ai_v[pl.ds(v * L, L)]
            nbr_bits = plsc.load_gather(rb_v, [nbr_idx])
            self_bits = plsc.load_gather(rb_v, [self_idx])
            vals.append(jnp.where((nbr_bits & self_bits) != 0,
                                  jnp.float32(1.0), jnp.float32(0.0)))
        for v in range(EV):
            o_v[pl.ds(eoff + v * L, L)] = vals[v]
        return carry

    edma.wait()
    with jax.named_scope("compute"):
        lax.fori_loop(0, NB, item, 0)
    with jax.named_scope("dma_out"):
        pltpu.sync_copy(o_v, out_hbm.at[pl.ds(base * EW, NB * EW)])


@jax.jit
def kernel(edges, rings):
    edges_i = edges.astype(jnp.int32).reshape(B * EW)
    rings_f = rings.reshape(B * RW)

    mesh = plsc.VectorSubcoreMesh(core_axis_name="c", subcore_axis_name="s",
                                  num_cores=NC, num_subcores=NS)
    run = pl.kernel(
        _find_ring_bonds_body,
        out_type=jax.ShapeDtypeStruct((B * EW,), jnp.float32),
        mesh=mesh,
        scratch_types=[
            pltpu.VMEM((NB * EW,), jnp.int32),
            pltpu.VMEM((NB * RW,), jnp.int32),
            pltpu.VMEM((NB * EW,), jnp.float32),
            pltpu.VMEM((EW,), jnp.int32),
            pltpu.VMEM((A,), jnp.int32),
            pltpu.VMEM((A,), jnp.int32),
            pltpu.VMEM((A,), jnp.int32),
            pltpu.SemaphoreType.DMA,
        ],
        compiler_params=pltpu.CompilerParams(needs_layout_passes=False,
                                             disable_bounds_checks=True),
    )
    out = run(edges_i, rings_f)
    return out.reshape(B, A, D, 1)


# speedup vs baseline: 5.6497x; 5.6497x over previous
import functools

import jax
import jax.numpy as jnp
from jax import lax
from jax.experimental import pallas as pl
from jax.experimental.pallas import tpu as pltpu
from jax.experimental.pallas import tpu_sc as plsc

B = 2048
A = 64
D = 6
R = 16
S = 8
L = 16
NC, NS = 2, 16
NW = NC * NS
NB = B // NW
EW = A * D
EV = EW // L
RW = R * S


def _find_ring_bonds_body(edges_hbm, rings_hbm, out_hbm,
                          e_v, r_v, o_v, ai_v, rb_v, ta_v, tb_v, sem):
    wid = lax.axis_index("c") * NS + lax.axis_index("s")
    base = wid * NB
    edma = pltpu.async_copy(edges_hbm.at[pl.ds(base, NB)], e_v, sem)
    with jax.named_scope("dma_in"):
        pltpu.sync_copy(rings_hbm.at[pl.ds(base, NB)], r_v)

    lane = lax.iota(jnp.int32, L)
    lo_mask = lane < S
    zeros = jnp.zeros((L,), jnp.int32)
    nib = jnp.full((L,), 0x11111111, jnp.int32)

    for v in range(EV):
        ai_v[pl.ds(v * L, L)] = (lane + v * L) // D

    def item(b, carry):
        for i in range(A // L):
            ta_v[pl.ds(i * L, L)] = zeros
            tb_v[pl.ds(i * L, L)] = zeros
        for r2 in range(0, R, 2):
            idx = r_v[b, pl.ds(r2 * S, L)]
            rr = r2 % 8
            val = jnp.where(lo_mask, 1 << (4 * rr), 1 << (4 * (rr + 1)))
            tab = ta_v if r2 < 8 else tb_v
            plsc.addupdate_scatter(tab, [idx], val)
        for i in range(A // L):
            va = ta_v[pl.ds(i * L, L)]
            vb = tb_v[pl.ds(i * L, L)]
            va = va | lax.shift_right_logical(va, 1)
            va = (va | lax.shift_right_logical(va, 2)) & nib
            vb = vb | lax.shift_right_logical(vb, 1)
            vb = (vb | lax.shift_right_logical(vb, 2)) & nib
            rb_v[pl.ds(i * L, L)] = va | (vb << 1)
        vals = []
        for v in range(EV):
            nbr_idx = e_v[b, pl.ds(v * L, L)]
            self_idx = ai_v[pl.ds(v * L, L)]
            nbr_bits = plsc.load_gather(rb_v, [nbr_idx])
            self_bits = plsc.load_gather(rb_v, [self_idx])
            vals.append(jnp.where((nbr_bits & self_bits) != 0,
                                  jnp.float32(1.0), jnp.float32(0.0)))
        for v in range(EV):
            o_v[b, pl.ds(v * L, L)] = vals[v]
        return carry

    edma.wait()
    with jax.named_scope("compute"):
        lax.fori_loop(0, NB, item, 0)
    with jax.named_scope("dma_out"):
        pltpu.sync_copy(o_v, out_hbm.at[pl.ds(base, NB)])


@jax.jit
def kernel(edges, rings):
    edges_i = edges.astype(jnp.int32).reshape(B, EW)
    rings_f = rings.reshape(B, RW)

    mesh = plsc.VectorSubcoreMesh(core_axis_name="c", subcore_axis_name="s",
                                  num_cores=NC, num_subcores=NS)
    run = pl.kernel(
        _find_ring_bonds_body,
        out_type=jax.ShapeDtypeStruct((B, EW), jnp.float32),
        mesh=mesh,
        scratch_types=[
            pltpu.VMEM((NB, EW), jnp.int32),
            pltpu.VMEM((NB, RW), jnp.int32),
            pltpu.VMEM((NB, EW), jnp.float32),
            pltpu.VMEM((EW,), jnp.int32),
            pltpu.VMEM((A,), jnp.int32),
            pltpu.VMEM((A,), jnp.int32),
            pltpu.VMEM((A,), jnp.int32),
            pltpu.SemaphoreType.DMA,
        ],
        compiler_params=pltpu.CompilerParams(needs_layout_passes=False,
                                             disable_bounds_checks=True),
    )
    out = run(edges_i, rings_f)
    return out.reshape(B, A, D, 1)
